# baseline (speedup 1.0000x reference)
import functools

import jax
import jax.numpy as jnp
from jax import lax
from jax.experimental import pallas as pl
from jax.experimental.pallas import tpu as pltpu
from jax.experimental.pallas import tpu_sc as plsc

B = 16384
D = 64
D2 = 2 * D
NC = 2
NS = 16
NW = NC * NS
PER_W = B // NW
SUB = 128
NSUB = PER_W // SUB

BLK = 2048


def _sc_gather_kernel():
    mesh = plsc.VectorSubcoreMesh(core_axis_name="c", subcore_axis_name="s")

    @functools.partial(
        pl.kernel,
        mesh=mesh,
        out_type=tuple(
            jax.ShapeDtypeStruct((B, D2), jnp.float32) for _ in range(4)),
        scratch_types=(
            pltpu.VMEM((NSUB, SUB), jnp.int32),
            pltpu.VMEM((NSUB, SUB), jnp.int32),
            pltpu.VMEM((SUB, D2), jnp.float32),
            pltpu.VMEM((SUB, D2), jnp.float32),
            pltpu.VMEM((SUB, D2), jnp.float32),
            pltpu.VMEM((SUB, D2), jnp.float32),
            pltpu.SemaphoreType.DMA,
            pltpu.SemaphoreType.DMA,
            pltpu.SemaphoreType.DMA,
            pltpu.SemaphoreType.DMA,
        ),
    )
    def sc_gather(uid_hbm, iid_hbm, gu_hbm, gi_hbm, mu_hbm, mi_hbm,
                  gu_out, gi_out, mu_out, mi_out,
                  u_idx, i_idx, gub, gib, mub, mib, s0, s1, s2, s3):
        wid = lax.axis_index("s") * NC + lax.axis_index("c")
        pltpu.sync_copy(uid_hbm.at[pl.ds(wid * NSUB, NSUB)], u_idx)
        pltpu.sync_copy(iid_hbm.at[pl.ds(wid * NSUB, NSUB)], i_idx)
        for j in range(NSUB):
            rbase = wid * PER_W + j * SUB
            c0 = pltpu.async_copy(gu_hbm.at[u_idx.at[j]], gub, s0)
            c1 = pltpu.async_copy(gi_hbm.at[i_idx.at[j]], gib, s1)
            c2 = pltpu.async_copy(mu_hbm.at[u_idx.at[j]], mub, s2)
            c3 = pltpu.async_copy(mi_hbm.at[i_idx.at[j]], mib, s3)
            c0.wait()
            pltpu.sync_copy(gub, gu_out.at[pl.ds(rbase, SUB)])
            c1.wait()
            pltpu.sync_copy(gib, gi_out.at[pl.ds(rbase, SUB)])
            c2.wait()
            pltpu.sync_copy(mub, mu_out.at[pl.ds(rbase, SUB)])
            c3.wait()
            pltpu.sync_copy(mib, mi_out.at[pl.ds(rbase, SUB)])

    return sc_gather


def _half(rows, par):
    return jnp.where(par, rows[:, D:], rows[:, :D])


def _tc_body(gu_ref, gi_ref, mu_ref, mi_ref, pu_ref, pi_ref,
             w1a, w1b, b1r, w2, b2r, w3, b3r, wgr, wmr, bor, out_ref):
    pu = pu_ref[...] != 0
    pi = pi_ref[...] != 0
    gu = _half(gu_ref[...], pu)
    gi = _half(gi_ref[...], pi)
    mu = _half(mu_ref[...], pu)
    mi = _half(mi_ref[...], pi)
    gmf = gu * gi
    h = jnp.dot(mu, w1a[...], preferred_element_type=jnp.float32)
    h = h + jnp.dot(mi, w1b[...], preferred_element_type=jnp.float32)
    h = jnp.maximum(h + b1r[...], 0.0)
    h = jnp.maximum(
        jnp.dot(h, w2[...], preferred_element_type=jnp.float32) + b2r[...], 0.0)
    h = jnp.maximum(
        jnp.dot(h, w3[...], preferred_element_type=jnp.float32) + b3r[...], 0.0)
    logit = (jnp.sum(gmf * wgr[...], axis=1)
             + jnp.sum(h * wmr[...], axis=1) + bor[0, 0])
    out_ref[...] = 1.0 + 4.0 * jax.nn.sigmoid(logit)


def _tc_mlp(gu, gi, mu, mi, pu, pi, W1a, W1b, b1, W2, b2, W3, b3, wg, wm, bo):
    grid = (B // BLK,)
    row_spec = pl.BlockSpec((BLK, D2), lambda i: (i, 0))
    par_spec = pl.BlockSpec((BLK, 1), lambda i: (i, 0))

    def full(shape):
        return pl.BlockSpec(shape, lambda i: tuple(0 for _ in shape))

    return pl.pallas_call(
        _tc_body,
        grid=grid,
        in_specs=[
            row_spec, row_spec, row_spec, row_spec, par_spec, par_spec,
            full((D, D)), full((D, D)), full((1, D)),
            full((D, 32)), full((1, 32)),
            full((32, 16)), full((1, 16)),
            full((1, D)), full((1, 16)), full((1, 1)),
        ],
        out_specs=pl.BlockSpec((BLK,), lambda i: (i,)),
        out_shape=jax.ShapeDtypeStruct((B,), jnp.float32),
    )(gu, gi, mu, mi, pu, pi, W1a, W1b, b1, W2, b2, W3, b3, wg, wm, bo)


def kernel(x, gmf_user, gmf_item, mlp_user, mlp_item,
           W1, b1, W2, b2, W3, b3, W_out, b_out):
    gu_t = gmf_user.reshape(500000, D2)
    gi_t = gmf_item.reshape(500000, D2)
    mu_t = mlp_user.reshape(500000, D2)
    mi_t = mlp_item.reshape(500000, D2)
    uid = x[:, 0]
    iid = x[:, 1]
    uh = (uid >> 1).reshape(NW * NSUB, SUB)
    ih = (iid >> 1).reshape(NW * NSUB, SUB)
    pu = (uid & 1).reshape(B, 1)
    pi = (iid & 1).reshape(B, 1)
    gu, gi, mu, mi = _sc_gather_kernel()(uh, ih, gu_t, gi_t, mu_t, mi_t)
    return _tc_mlp(
        gu, gi, mu, mi, pu, pi,
        W1[:D], W1[D:], b1.reshape(1, D),
        W2, b2.reshape(1, 32), W3, b3.reshape(1, 16),
        W_out[:D, 0].reshape(1, D), W_out[D:, 0].reshape(1, 16),
        b_out.reshape(1, 1))


# speedup vs baseline: 1.5879x; 1.5879x over previous
import functools

import jax
import jax.numpy as jnp
from jax import lax
from jax.experimental import pallas as pl
from jax.experimental.pallas import tpu as pltpu
from jax.experimental.pallas import tpu_sc as plsc

B = 16384
D = 64
D2 = 2 * D
V = 1000000
NC = 2
NS = 16
NW = NC * NS
PER_W = B // NW
SUB = 128
NSUB = PER_W // SUB

CH = 2048
NCH = -(-V // CH)
BLK = 2048


def _transform_body(a_ref, b_ref, wa_ref, wb_ref, out_ref):
    dn = (((0,), (0,)), ((), ()))
    a = lax.dot_general(a_ref[...], wa_ref[...], dn,
                        preferred_element_type=jnp.float32)
    b = lax.dot_general(b_ref[...], wb_ref[...], dn,
                        preferred_element_type=jnp.float32)
    out_ref[...] = jnp.concatenate([a, b], axis=1)


def _tc_transform(tab_a_t, tab_b_t, wa, wb):
    return pl.pallas_call(
        _transform_body,
        grid=(NCH,),
        in_specs=[
            pl.BlockSpec((D, CH), lambda i: (0, i)),
            pl.BlockSpec((D, CH), lambda i: (0, i)),
            pl.BlockSpec((D, D), lambda i: (0, 0)),
            pl.BlockSpec((D, D), lambda i: (0, 0)),
        ],
        out_specs=pl.BlockSpec((CH, D2), lambda i: (i, 0)),
        out_shape=jax.ShapeDtypeStruct((V, D2), jnp.float32),
    )(tab_a_t, tab_b_t, wa, wb)


def _sc_gather_kernel():
    mesh = plsc.VectorSubcoreMesh(core_axis_name="c", subcore_axis_name="s")

    @functools.partial(
        pl.kernel,
        mesh=mesh,
        out_type=(
            jax.ShapeDtypeStruct((B, D2), jnp.float32),
            jax.ShapeDtypeStruct((B, D2), jnp.float32),
        ),
        scratch_types=(
            pltpu.VMEM((NSUB, SUB), jnp.int32),
            pltpu.VMEM((NSUB, SUB), jnp.int32),
            pltpu.VMEM((SUB, D2), jnp.float32),
            pltpu.VMEM((SUB, D2), jnp.float32),
            pltpu.SemaphoreType.DMA,
            pltpu.SemaphoreType.DMA,
        ),
    )
    def sc_gather(uid_hbm, iid_hbm, ut_hbm, it_hbm,
                  u_out, i_out,
                  u_idx, i_idx, ub, ib, s0, s1):
        wid = lax.axis_index("s") * NC + lax.axis_index("c")
        pltpu.sync_copy(uid_hbm.at[pl.ds(wid * NSUB, NSUB)], u_idx)
        pltpu.sync_copy(iid_hbm.at[pl.ds(wid * NSUB, NSUB)], i_idx)
        for j in range(NSUB):
            rbase = wid * PER_W + j * SUB
            c0 = pltpu.async_copy(ut_hbm.at[u_idx.at[j]], ub, s0)
            c1 = pltpu.async_copy(it_hbm.at[i_idx.at[j]], ib, s1)
            c0.wait()
            pltpu.sync_copy(ub, u_out.at[pl.ds(rbase, SUB)])
            c1.wait()
            pltpu.sync_copy(ib, i_out.at[pl.ds(rbase, SUB)])

    return sc_gather


def _final_body(u_ref, i_ref, b1r, w2, b2r, w3, b3r, wmr, bor, out_ref):
    uu = u_ref[...]
    ii = i_ref[...]
    gmf_logit = jnp.sum(uu[:, :D] * ii[:, :D], axis=1)
    h = jnp.maximum(uu[:, D:] + ii[:, D:] + b1r[...], 0.0)
    h = jnp.maximum(
        jnp.dot(h, w2[...], preferred_element_type=jnp.float32) + b2r[...], 0.0)
    h = jnp.maximum(
        jnp.dot(h, w3[...], preferred_element_type=jnp.float32) + b3r[...], 0.0)
    logit = gmf_logit + jnp.sum(h * wmr[...], axis=1) + bor[0, 0]
    out_ref[...] = 1.0 + 4.0 * jax.nn.sigmoid(logit)


def _tc_final(u_rows, i_rows, b1, W2, b2, W3, b3, wm, bo):
    grid = (B // BLK,)
    row_spec = pl.BlockSpec((BLK, D2), lambda i: (i, 0))

    def full(shape):
        return pl.BlockSpec(shape, lambda i: tuple(0 for _ in shape))

    return pl.pallas_call(
        _final_body,
        grid=grid,
        in_specs=[
            row_spec, row_spec,
            full((1, D)),
            full((D, 32)), full((1, 32)),
            full((32, 16)), full((1, 16)),
            full((1, 16)), full((1, 1)),
        ],
        out_specs=pl.BlockSpec((BLK,), lambda i: (i,)),
        out_shape=jax.ShapeDtypeStruct((B,), jnp.float32),
    )(u_rows, i_rows, b1, W2, b2, W3, b3, wm, bo)


def kernel(x, gmf_user, gmf_item, mlp_user, mlp_item,
           W1, b1, W2, b2, W3, b3, W_out, b_out):
    diag_wg = jnp.diag(W_out[:D, 0])
    eye = jnp.eye(D, dtype=jnp.float32)
    u_tab = _tc_transform(gmf_user.T, mlp_user.T, diag_wg, W1[:D])
    i_tab = _tc_transform(gmf_item.T, mlp_item.T, eye, W1[D:])
    uid = x[:, 0].reshape(NW * NSUB, SUB)
    iid = x[:, 1].reshape(NW * NSUB, SUB)
    u_rows, i_rows = _sc_gather_kernel()(uid, iid, u_tab, i_tab)
    return _tc_final(
        u_rows, i_rows,
        b1.reshape(1, D), W2, b2.reshape(1, 32), W3, b3.reshape(1, 16),
        W_out[D:, 0].reshape(1, 16), b_out.reshape(1, 1))


# speedup vs baseline: 1.7295x; 1.0891x over previous
import functools

import jax
import jax.numpy as jnp
from jax import lax
from jax.experimental import pallas as pl
from jax.experimental.pallas import tpu as pltpu
from jax.experimental.pallas import tpu_sc as plsc

B = 16384
D = 64
D2 = 2 * D
V = 1000000
NC = 2
NS = 16
NW = NC * NS
PER_W = B // NW
SUB = 128
NSUB = PER_W // SUB

CH = 2048
NCH = -(-V // CH)
BLK = 2048


def _transform_body(a_ref, b_ref, wa_ref, wb_ref, out_ref):
    dn = (((0,), (0,)), ((), ()))
    a = lax.dot_general(a_ref[...].astype(jnp.bfloat16), wa_ref[...], dn,
                        preferred_element_type=jnp.float32)
    b = lax.dot_general(b_ref[...].astype(jnp.bfloat16), wb_ref[...], dn,
                        preferred_element_type=jnp.float32)
    out_ref[...] = jnp.concatenate([a, b], axis=1)


def _tc_transform(tab_a_t, tab_b_t, wa, wb):
    return pl.pallas_call(
        _transform_body,
        grid=(NCH,),
        in_specs=[
            pl.BlockSpec((D, CH), lambda i: (0, i)),
            pl.BlockSpec((D, CH), lambda i: (0, i)),
            pl.BlockSpec((D, D), lambda i: (0, 0)),
            pl.BlockSpec((D, D), lambda i: (0, 0)),
        ],
        out_specs=pl.BlockSpec((CH, D2), lambda i: (i, 0)),
        out_shape=jax.ShapeDtypeStruct((V, D2), jnp.float32),
    )(tab_a_t, tab_b_t, wa, wb)


def _sc_gather_kernel():
    mesh = plsc.VectorSubcoreMesh(core_axis_name="c", subcore_axis_name="s")

    @functools.partial(
        pl.kernel,
        mesh=mesh,
        out_type=(
            jax.ShapeDtypeStruct((B, D2), jnp.float32),
            jax.ShapeDtypeStruct((B, D2), jnp.float32),
        ),
        scratch_types=(
            pltpu.VMEM((NSUB, SUB), jnp.int32),
            pltpu.VMEM((NSUB, SUB), jnp.int32),
            pltpu.VMEM((SUB, D2), jnp.float32),
            pltpu.VMEM((SUB, D2), jnp.float32),
            pltpu.SemaphoreType.DMA,
            pltpu.SemaphoreType.DMA,
        ),
    )
    def sc_gather(uid_hbm, iid_hbm, ut_hbm, it_hbm,
                  u_out, i_out,
                  u_idx, i_idx, ub, ib, s0, s1):
        wid = lax.axis_index("s") * NC + lax.axis_index("c")
        pltpu.sync_copy(uid_hbm.at[pl.ds(wid * NSUB, NSUB)], u_idx)
        pltpu.sync_copy(iid_hbm.at[pl.ds(wid * NSUB, NSUB)], i_idx)
        for j in range(NSUB):
            rbase = wid * PER_W + j * SUB
            c0 = pltpu.async_copy(ut_hbm.at[u_idx.at[j]], ub, s0)
            c1 = pltpu.async_copy(it_hbm.at[i_idx.at[j]], ib, s1)
            c0.wait()
            pltpu.sync_copy(ub, u_out.at[pl.ds(rbase, SUB)])
            c1.wait()
            pltpu.sync_copy(ib, i_out.at[pl.ds(rbase, SUB)])

    return sc_gather


def _final_body(u_ref, i_ref, b1r, w2, b2r, w3, b3r, wmr, bor, out_ref):
    uu = u_ref[...]
    ii = i_ref[...]
    gmf_logit = jnp.sum(uu[:, :D] * ii[:, :D], axis=1)
    h = jnp.maximum(uu[:, D:] + ii[:, D:] + b1r[...], 0.0)
    h = jnp.maximum(
        jnp.dot(h, w2[...], preferred_element_type=jnp.float32) + b2r[...], 0.0)
    h = jnp.maximum(
        jnp.dot(h, w3[...], preferred_element_type=jnp.float32) + b3r[...], 0.0)
    logit = gmf_logit + jnp.sum(h * wmr[...], axis=1) + bor[0, 0]
    out_ref[...] = 1.0 + 4.0 * jax.nn.sigmoid(logit)


def _tc_final(u_rows, i_rows, b1, W2, b2, W3, b3, wm, bo):
    grid = (B // BLK,)
    row_spec = pl.BlockSpec((BLK, D2), lambda i: (i, 0))

    def full(shape):
        return pl.BlockSpec(shape, lambda i: tuple(0 for _ in shape))

    return pl.pallas_call(
        _final_body,
        grid=grid,
        in_specs=[
            row_spec, row_spec,
            full((1, D)),
            full((D, 32)), full((1, 32)),
            full((32, 16)), full((1, 16)),
            full((1, 16)), full((1, 1)),
        ],
        out_specs=pl.BlockSpec((BLK,), lambda i: (i,)),
        out_shape=jax.ShapeDtypeStruct((B,), jnp.float32),
    )(u_rows, i_rows, b1, W2, b2, W3, b3, wm, bo)


def kernel(x, gmf_user, gmf_item, mlp_user, mlp_item,
           W1, b1, W2, b2, W3, b3, W_out, b_out):
    diag_wg = jnp.diag(W_out[:D, 0]).astype(jnp.bfloat16)
    eye = jnp.eye(D, dtype=jnp.bfloat16)
    u_tab = _tc_transform(gmf_user.T, mlp_user.T, diag_wg,
                          W1[:D].astype(jnp.bfloat16))
    i_tab = _tc_transform(gmf_item.T, mlp_item.T, eye,
                          W1[D:].astype(jnp.bfloat16))
    uid = x[:, 0].reshape(NW * NSUB, SUB)
    iid = x[:, 1].reshape(NW * NSUB, SUB)
    u_rows, i_rows = _sc_gather_kernel()(uid, iid, u_tab, i_tab)
    return _tc_final(
        u_rows, i_rows,
        b1.reshape(1, D), W2, b2.reshape(1, 32), W3, b3.reshape(1, 16),
        W_out[D:, 0].reshape(1, 16), b_out.reshape(1, 1))


# speedup vs baseline: 2.6443x; 1.5290x over previous
import functools

import jax
import jax.numpy as jnp
from jax import lax
from jax.experimental import pallas as pl
from jax.experimental.pallas import tpu as pltpu
from jax.experimental.pallas import tpu_sc as plsc

B = 16384
D = 64
D2 = 2 * D
V = 1000000
NC = 2
NS = 16
NW = NC * NS
PER_W = B // NW
SUB = 128
NSUB = PER_W // SUB

CH = 8192
NCH = -(-V // CH)
BLK = 2048


def _transform_body(a_ref, b_ref, wa_ref, wb_ref, out_ref):
    dn = (((0,), (0,)), ((), ()))
    a = lax.dot_general(a_ref[...].astype(jnp.bfloat16), wa_ref[...], dn,
                        preferred_element_type=jnp.float32)
    b = lax.dot_general(b_ref[...].astype(jnp.bfloat16), wb_ref[...], dn,
                        preferred_element_type=jnp.float32)
    out_ref[...] = jnp.concatenate([a, b], axis=1)


def _tc_transform(tab_a_t, tab_b_t, wa, wb):
    return pl.pallas_call(
        _transform_body,
        grid=(NCH,),
        in_specs=[
            pl.BlockSpec((D, CH), lambda i: (0, i)),
            pl.BlockSpec((D, CH), lambda i: (0, i)),
            pl.BlockSpec((D, D), lambda i: (0, 0)),
            pl.BlockSpec((D, D), lambda i: (0, 0)),
        ],
        out_specs=pl.BlockSpec((CH, D2), lambda i: (i, 0)),
        out_shape=jax.ShapeDtypeStruct((V, D2), jnp.float32),
    )(tab_a_t, tab_b_t, wa, wb)


def _sc_gather_kernel():
    mesh = plsc.VectorSubcoreMesh(core_axis_name="c", subcore_axis_name="s")

    @functools.partial(
        pl.kernel,
        mesh=mesh,
        out_type=(
            jax.ShapeDtypeStruct((B, D2), jnp.float32),
            jax.ShapeDtypeStruct((B, D2), jnp.float32),
        ),
        scratch_types=(
            pltpu.VMEM((NSUB, SUB), jnp.int32),
            pltpu.VMEM((NSUB, SUB), jnp.int32),
            pltpu.VMEM((SUB, D2), jnp.float32),
            pltpu.VMEM((SUB, D2), jnp.float32),
            pltpu.SemaphoreType.DMA,
            pltpu.SemaphoreType.DMA,
        ),
    )
    def sc_gather(uid_hbm, iid_hbm, ut_hbm, it_hbm,
                  u_out, i_out,
                  u_idx, i_idx, ub, ib, s0, s1):
        wid = lax.axis_index("s") * NC + lax.axis_index("c")
        pltpu.sync_copy(uid_hbm.at[pl.ds(wid * NSUB, NSUB)], u_idx)
        pltpu.sync_copy(iid_hbm.at[pl.ds(wid * NSUB, NSUB)], i_idx)
        for j in range(NSUB):
            rbase = wid * PER_W + j * SUB
            c0 = pltpu.async_copy(ut_hbm.at[u_idx.at[j]], ub, s0)
            c1 = pltpu.async_copy(it_hbm.at[i_idx.at[j]], ib, s1)
            c0.wait()
            pltpu.sync_copy(ub, u_out.at[pl.ds(rbase, SUB)])
            c1.wait()
            pltpu.sync_copy(ib, i_out.at[pl.ds(rbase, SUB)])

    return sc_gather


def _final_body(u_ref, i_ref, b1r, w2, b2r, w3, b3r, wmr, bor, out_ref):
    uu = u_ref[...]
    ii = i_ref[...]
    gmf_logit = jnp.sum(uu[:, :D] * ii[:, :D], axis=1)
    h = jnp.maximum(uu[:, D:] + ii[:, D:] + b1r[...], 0.0)
    h = jnp.maximum(
        jnp.dot(h, w2[...], preferred_element_type=jnp.float32) + b2r[...], 0.0)
    h = jnp.maximum(
        jnp.dot(h, w3[...], preferred_element_type=jnp.float32) + b3r[...], 0.0)
    logit = gmf_logit + jnp.sum(h * wmr[...], axis=1) + bor[0, 0]
    out_ref[...] = 1.0 + 4.0 * jax.nn.sigmoid(logit)


def _tc_final(u_rows, i_rows, b1, W2, b2, W3, b3, wm, bo):
    grid = (B // BLK,)
    row_spec = pl.BlockSpec((BLK, D2), lambda i: (i, 0))

    def full(shape):
        return pl.BlockSpec(shape, lambda i: tuple(0 for _ in shape))

    return pl.pallas_call(
        _final_body,
        grid=grid,
        in_specs=[
            row_spec, row_spec,
            full((1, D)),
            full((D, 32)), full((1, 32)),
            full((32, 16)), full((1, 16)),
            full((1, 16)), full((1, 1)),
        ],
        out_specs=pl.BlockSpec((BLK,), lambda i: (i,)),
        out_shape=jax.ShapeDtypeStruct((B,), jnp.float32),
    )(u_rows, i_rows, b1, W2, b2, W3, b3, wm, bo)


def kernel(x, gmf_user, gmf_item, mlp_user, mlp_item,
           W1, b1, W2, b2, W3, b3, W_out, b_out):
    diag_wg = jnp.diag(W_out[:D, 0]).astype(jnp.bfloat16)
    eye = jnp.eye(D, dtype=jnp.bfloat16)
    u_tab = _tc_transform(gmf_user.T, mlp_user.T, diag_wg,
                          W1[:D].astype(jnp.bfloat16))
    i_tab = _tc_transform(gmf_item.T, mlp_item.T, eye,
                          W1[D:].astype(jnp.bfloat16))
    uid = x[:, 0].reshape(NW * NSUB, SUB)
    iid = x[:, 1].reshape(NW * NSUB, SUB)
    u_rows, i_rows = _sc_gather_kernel()(uid, iid, u_tab, i_tab)
    return _tc_final(
        u_rows, i_rows,
        b1.reshape(1, D), W2, b2.reshape(1, 32), W3, b3.reshape(1, 16),
        W_out[D:, 0].reshape(1, 16), b_out.reshape(1, 1))


# speedup vs baseline: 2.9213x; 1.1047x over previous
import functools

import jax
import jax.numpy as jnp
from jax import lax
from jax.experimental import pallas as pl
from jax.experimental.pallas import tpu as pltpu
from jax.experimental.pallas import tpu_sc as plsc

B = 16384
D = 64
D2 = 2 * D
V = 1000000
NC = 2
NS = 16
NW = NC * NS
PER_W = B // NW
SUB = 128
NSUB = PER_W // SUB

CH = 16384
NCH = -(-V // CH)
BLK = 2048


def _transform_body(a_ref, b_ref, wa_ref, wb_ref, out_ref):
    dn = (((0,), (0,)), ((), ()))
    a = lax.dot_general(a_ref[...].astype(jnp.bfloat16), wa_ref[...], dn,
                        preferred_element_type=jnp.float32)
    b = lax.dot_general(b_ref[...].astype(jnp.bfloat16), wb_ref[...], dn,
                        preferred_element_type=jnp.float32)
    out_ref[...] = jnp.concatenate([a, b], axis=1)


def _tc_transform(tab_a_t, tab_b_t, wa, wb):
    return pl.pallas_call(
        _transform_body,
        grid=(NCH,),
        in_specs=[
            pl.BlockSpec((D, CH), lambda i: (0, i)),
            pl.BlockSpec((D, CH), lambda i: (0, i)),
            pl.BlockSpec((D, D), lambda i: (0, 0)),
            pl.BlockSpec((D, D), lambda i: (0, 0)),
        ],
        out_specs=pl.BlockSpec((CH, D2), lambda i: (i, 0)),
        out_shape=jax.ShapeDtypeStruct((V, D2), jnp.float32),
    )(tab_a_t, tab_b_t, wa, wb)


def _sc_gather_kernel():
    mesh = plsc.VectorSubcoreMesh(core_axis_name="c", subcore_axis_name="s")

    @functools.partial(
        pl.kernel,
        mesh=mesh,
        out_type=(
            jax.ShapeDtypeStruct((B, D2), jnp.float32),
            jax.ShapeDtypeStruct((B, D2), jnp.float32),
        ),
        scratch_types=(
            pltpu.VMEM((NSUB, SUB), jnp.int32),
            pltpu.VMEM((NSUB, SUB), jnp.int32),
            pltpu.VMEM((SUB, D2), jnp.float32),
            pltpu.VMEM((SUB, D2), jnp.float32),
            pltpu.SemaphoreType.DMA,
            pltpu.SemaphoreType.DMA,
        ),
    )
    def sc_gather(uid_hbm, iid_hbm, ut_hbm, it_hbm,
                  u_out, i_out,
                  u_idx, i_idx, ub, ib, s0, s1):
        wid = lax.axis_index("s") * NC + lax.axis_index("c")
        pltpu.sync_copy(uid_hbm.at[pl.ds(wid * NSUB, NSUB)], u_idx)
        pltpu.sync_copy(iid_hbm.at[pl.ds(wid * NSUB, NSUB)], i_idx)
        for j in range(NSUB):
            rbase = wid * PER_W + j * SUB
            c0 = pltpu.async_copy(ut_hbm.at[u_idx.at[j]], ub, s0)
            c1 = pltpu.async_copy(it_hbm.at[i_idx.at[j]], ib, s1)
            c0.wait()
            pltpu.sync_copy(ub, u_out.at[pl.ds(rbase, SUB)])
            c1.wait()
            pltpu.sync_copy(ib, i_out.at[pl.ds(rbase, SUB)])

    return sc_gather


def _final_body(u_ref, i_ref, b1r, w2, b2r, w3, b3r, wmr, bor, out_ref):
    uu = u_ref[...]
    ii = i_ref[...]
    gmf_logit = jnp.sum(uu[:, :D] * ii[:, :D], axis=1)
    h = jnp.maximum(uu[:, D:] + ii[:, D:] + b1r[...], 0.0)
    h = jnp.maximum(
        jnp.dot(h, w2[...], preferred_element_type=jnp.float32) + b2r[...], 0.0)
    h = jnp.maximum(
        jnp.dot(h, w3[...], preferred_element_type=jnp.float32) + b3r[...], 0.0)
    logit = gmf_logit + jnp.sum(h * wmr[...], axis=1) + bor[0, 0]
    out_ref[...] = 1.0 + 4.0 * jax.nn.sigmoid(logit)


def _tc_final(u_rows, i_rows, b1, W2, b2, W3, b3, wm, bo):
    grid = (B // BLK,)
    row_spec = pl.BlockSpec((BLK, D2), lambda i: (i, 0))

    def full(shape):
        return pl.BlockSpec(shape, lambda i: tuple(0 for _ in shape))

    return pl.pallas_call(
        _final_body,
        grid=grid,
        in_specs=[
            row_spec, row_spec,
            full((1, D)),
            full((D, 32)), full((1, 32)),
            full((32, 16)), full((1, 16)),
            full((1, 16)), full((1, 1)),
        ],
        out_specs=pl.BlockSpec((BLK,), lambda i: (i,)),
        out_shape=jax.ShapeDtypeStruct((B,), jnp.float32),
    )(u_rows, i_rows, b1, W2, b2, W3, b3, wm, bo)


def kernel(x, gmf_user, gmf_item, mlp_user, mlp_item,
           W1, b1, W2, b2, W3, b3, W_out, b_out):
    diag_wg = jnp.diag(W_out[:D, 0]).astype(jnp.bfloat16)
    eye = jnp.eye(D, dtype=jnp.bfloat16)
    u_tab = _tc_transform(gmf_user.T, mlp_user.T, diag_wg,
                          W1[:D].astype(jnp.bfloat16))
    i_tab = _tc_transform(gmf_item.T, mlp_item.T, eye,
                          W1[D:].astype(jnp.bfloat16))
    uid = x[:, 0].reshape(NW * NSUB, SUB)
    iid = x[:, 1].reshape(NW * NSUB, SUB)
    u_rows, i_rows = _sc_gather_kernel()(uid, iid, u_tab, i_tab)
    return _tc_final(
        u_rows, i_rows,
        b1.reshape(1, D), W2, b2.reshape(1, 32), W3, b3.reshape(1, 16),
        W_out[D:, 0].reshape(1, 16), b_out.reshape(1, 1))


# speedup vs baseline: 3.4613x; 1.1849x over previous
import functools

import jax
import jax.numpy as jnp
from jax import lax
from jax.experimental import pallas as pl
from jax.experimental.pallas import tpu as pltpu
from jax.experimental.pallas import tpu_sc as plsc

B = 16384
D = 64
D2 = 2 * D
V = 1000000
NC = 2
NS = 16
NW = NC * NS
PER_W = B // NW
SUB = 128
NSUB = PER_W // SUB

CH = 16384
NCH = -(-V // CH)
BLK = 2048


def _transform_body(a_ref, b_ref, wa_ref, wb_ref, out_ref):
    dn = (((0,), (0,)), ((), ()))
    a = lax.dot_general(a_ref[...].astype(jnp.bfloat16), wa_ref[...], dn,
                        preferred_element_type=jnp.float32)
    b = lax.dot_general(b_ref[...].astype(jnp.bfloat16), wb_ref[...], dn,
                        preferred_element_type=jnp.float32)
    rows = jnp.concatenate([a, b], axis=1).astype(jnp.bfloat16)
    out_ref[...] = pltpu.bitcast(rows, jnp.int32)


def _tc_transform(tab_a_t, tab_b_t, wa, wb):
    return pl.pallas_call(
        _transform_body,
        grid=(NCH,),
        in_specs=[
            pl.BlockSpec((D, CH), lambda i: (0, i)),
            pl.BlockSpec((D, CH), lambda i: (0, i)),
            pl.BlockSpec((D, D), lambda i: (0, 0)),
            pl.BlockSpec((D, D), lambda i: (0, 0)),
        ],
        out_specs=pl.BlockSpec((CH // 2, D2), lambda i: (i, 0)),
        out_shape=jax.ShapeDtypeStruct((V // 2, D2), jnp.int32),
    )(tab_a_t, tab_b_t, wa, wb)


def _sc_gather_kernel():
    mesh = plsc.VectorSubcoreMesh(core_axis_name="c", subcore_axis_name="s")

    @functools.partial(
        pl.kernel,
        mesh=mesh,
        out_type=(
            jax.ShapeDtypeStruct((B, D2), jnp.int32),
            jax.ShapeDtypeStruct((B, D2), jnp.int32),
        ),
        scratch_types=(
            pltpu.VMEM((NSUB, SUB), jnp.int32),
            pltpu.VMEM((NSUB, SUB), jnp.int32),
            pltpu.VMEM((SUB, D2), jnp.int32),
            pltpu.VMEM((SUB, D2), jnp.int32),
            pltpu.SemaphoreType.DMA,
            pltpu.SemaphoreType.DMA,
        ),
    )
    def sc_gather(uid_hbm, iid_hbm, ut_hbm, it_hbm,
                  u_out, i_out,
                  u_idx, i_idx, ub, ib, s0, s1):
        wid = lax.axis_index("s") * NC + lax.axis_index("c")
        pltpu.sync_copy(uid_hbm.at[pl.ds(wid * NSUB, NSUB)], u_idx)
        pltpu.sync_copy(iid_hbm.at[pl.ds(wid * NSUB, NSUB)], i_idx)
        for j in range(NSUB):
            rbase = wid * PER_W + j * SUB
            c0 = pltpu.async_copy(ut_hbm.at[u_idx.at[j]], ub, s0)
            c1 = pltpu.async_copy(it_hbm.at[i_idx.at[j]], ib, s1)
            c0.wait()
            pltpu.sync_copy(ub, u_out.at[pl.ds(rbase, SUB)])
            c1.wait()
            pltpu.sync_copy(ib, i_out.at[pl.ds(rbase, SUB)])

    return sc_gather


def _unpack(w, par):
    hi = w & jnp.int32(-65536)
    lo = w << 16
    return lax.bitcast_convert_type(jnp.where(par, hi, lo), jnp.float32)


def _final_body(u_ref, i_ref, pu_ref, pi_ref,
                b1r, w2, b2r, w3, b3r, wmr, bor, out_ref):
    uu = _unpack(u_ref[...], pu_ref[...] != 0)
    ii = _unpack(i_ref[...], pi_ref[...] != 0)
    gmf_logit = jnp.sum(uu[:, :D] * ii[:, :D], axis=1)
    h = jnp.maximum(uu[:, D:] + ii[:, D:] + b1r[...], 0.0)
    h = jnp.maximum(
        jnp.dot(h, w2[...], preferred_element_type=jnp.float32) + b2r[...], 0.0)
    h = jnp.maximum(
        jnp.dot(h, w3[...], preferred_element_type=jnp.float32) + b3r[...], 0.0)
    logit = gmf_logit + jnp.sum(h * wmr[...], axis=1) + bor[0, 0]
    out_ref[...] = 1.0 + 4.0 * jax.nn.sigmoid(logit)


def _tc_final(u_rows, i_rows, pu, pi, b1, W2, b2, W3, b3, wm, bo):
    grid = (B // BLK,)
    row_spec = pl.BlockSpec((BLK, D2), lambda i: (i, 0))
    par_spec = pl.BlockSpec((BLK, 1), lambda i: (i, 0))

    def full(shape):
        return pl.BlockSpec(shape, lambda i: tuple(0 for _ in shape))

    return pl.pallas_call(
        _final_body,
        grid=grid,
        in_specs=[
            row_spec, row_spec, par_spec, par_spec,
            full((1, D)),
            full((D, 32)), full((1, 32)),
            full((32, 16)), full((1, 16)),
            full((1, 16)), full((1, 1)),
        ],
        out_specs=pl.BlockSpec((BLK,), lambda i: (i,)),
        out_shape=jax.ShapeDtypeStruct((B,), jnp.float32),
    )(u_rows, i_rows, pu, pi, b1, W2, b2, W3, b3, wm, bo)


def kernel(x, gmf_user, gmf_item, mlp_user, mlp_item,
           W1, b1, W2, b2, W3, b3, W_out, b_out):
    diag_wg = jnp.diag(W_out[:D, 0]).astype(jnp.bfloat16)
    eye = jnp.eye(D, dtype=jnp.bfloat16)
    u_tab = _tc_transform(gmf_user.T, mlp_user.T, diag_wg,
                          W1[:D].astype(jnp.bfloat16))
    i_tab = _tc_transform(gmf_item.T, mlp_item.T, eye,
                          W1[D:].astype(jnp.bfloat16))
    uid = x[:, 0]
    iid = x[:, 1]
    uh = (uid >> 1).reshape(NW * NSUB, SUB)
    ih = (iid >> 1).reshape(NW * NSUB, SUB)
    pu = (uid & 1).reshape(B, 1)
    pi = (iid & 1).reshape(B, 1)
    u_rows, i_rows = _sc_gather_kernel()(uh, ih, u_tab, i_tab)
    return _tc_final(
        u_rows, i_rows, pu, pi,
        b1.reshape(1, D), W2, b2.reshape(1, 32), W3, b3.reshape(1, 16),
        W_out[D:, 0].reshape(1, 16), b_out.reshape(1, 1))


# speedup vs baseline: 3.6292x; 1.0485x over previous
import functools

import jax
import jax.numpy as jnp
from jax import lax
from jax.experimental import pallas as pl
from jax.experimental.pallas import tpu as pltpu
from jax.experimental.pallas import tpu_sc as plsc

B = 16384
D = 64
D2 = 2 * D
V = 1000000
NC = 2
NS = 16
NW = NC * NS
PER_W = B // NW
SUB = 128
NSUB = PER_W // SUB

CH = 24576
NCH = -(-V // CH)
BLK = 2048


def _transform_body(a_ref, b_ref, wa_ref, wb_ref, out_ref):
    dn = (((0,), (0,)), ((), ()))
    a = lax.dot_general(a_ref[...].astype(jnp.bfloat16), wa_ref[...], dn,
                        preferred_element_type=jnp.float32)
    b = lax.dot_general(b_ref[...].astype(jnp.bfloat16), wb_ref[...], dn,
                        preferred_element_type=jnp.float32)
    rows = jnp.concatenate([a, b], axis=1).astype(jnp.bfloat16)
    out_ref[...] = pltpu.bitcast(rows, jnp.int32)


def _tc_transform(tab_a_t, tab_b_t, wa, wb):
    return pl.pallas_call(
        _transform_body,
        grid=(NCH,),
        in_specs=[
            pl.BlockSpec((D, CH), lambda i: (0, i)),
            pl.BlockSpec((D, CH), lambda i: (0, i)),
            pl.BlockSpec((D, D), lambda i: (0, 0)),
            pl.BlockSpec((D, D), lambda i: (0, 0)),
        ],
        out_specs=pl.BlockSpec((CH // 2, D2), lambda i: (i, 0)),
        out_shape=jax.ShapeDtypeStruct((V // 2, D2), jnp.int32),
    )(tab_a_t, tab_b_t, wa, wb)


def _sc_gather_kernel():
    mesh = plsc.VectorSubcoreMesh(core_axis_name="c", subcore_axis_name="s")

    @functools.partial(
        pl.kernel,
        mesh=mesh,
        out_type=(
            jax.ShapeDtypeStruct((B, D2), jnp.int32),
            jax.ShapeDtypeStruct((B, D2), jnp.int32),
        ),
        scratch_types=(
            pltpu.VMEM((NSUB, SUB), jnp.int32),
            pltpu.VMEM((NSUB, SUB), jnp.int32),
            pltpu.VMEM((SUB, D2), jnp.int32),
            pltpu.VMEM((SUB, D2), jnp.int32),
            pltpu.SemaphoreType.DMA,
            pltpu.SemaphoreType.DMA,
        ),
    )
    def sc_gather(uid_hbm, iid_hbm, ut_hbm, it_hbm,
                  u_out, i_out,
                  u_idx, i_idx, ub, ib, s0, s1):
        wid = lax.axis_index("s") * NC + lax.axis_index("c")
        pltpu.sync_copy(uid_hbm.at[pl.ds(wid * NSUB, NSUB)], u_idx)
        pltpu.sync_copy(iid_hbm.at[pl.ds(wid * NSUB, NSUB)], i_idx)
        for j in range(NSUB):
            rbase = wid * PER_W + j * SUB
            c0 = pltpu.async_copy(ut_hbm.at[u_idx.at[j]], ub, s0)
            c1 = pltpu.async_copy(it_hbm.at[i_idx.at[j]], ib, s1)
            c0.wait()
            pltpu.sync_copy(ub, u_out.at[pl.ds(rbase, SUB)])
            c1.wait()
            pltpu.sync_copy(ib, i_out.at[pl.ds(rbase, SUB)])

    return sc_gather


def _unpack(w, par):
    hi = w & jnp.int32(-65536)
    lo = w << 16
    return lax.bitcast_convert_type(jnp.where(par, hi, lo), jnp.float32)


def _final_body(u_ref, i_ref, pu_ref, pi_ref,
                b1r, w2, b2r, w3, b3r, wmr, bor, out_ref):
    uu = _unpack(u_ref[...], pu_ref[...] != 0)
    ii = _unpack(i_ref[...], pi_ref[...] != 0)
    gmf_logit = jnp.sum(uu[:, :D] * ii[:, :D], axis=1)
    h = jnp.maximum(uu[:, D:] + ii[:, D:] + b1r[...], 0.0)
    h = jnp.maximum(
        jnp.dot(h, w2[...], preferred_element_type=jnp.float32) + b2r[...], 0.0)
    h = jnp.maximum(
        jnp.dot(h, w3[...], preferred_element_type=jnp.float32) + b3r[...], 0.0)
    logit = gmf_logit + jnp.sum(h * wmr[...], axis=1) + bor[0, 0]
    out_ref[...] = 1.0 + 4.0 * jax.nn.sigmoid(logit)


def _tc_final(u_rows, i_rows, pu, pi, b1, W2, b2, W3, b3, wm, bo):
    grid = (B // BLK,)
    row_spec = pl.BlockSpec((BLK, D2), lambda i: (i, 0))
    par_spec = pl.BlockSpec((BLK, 1), lambda i: (i, 0))

    def full(shape):
        return pl.BlockSpec(shape, lambda i: tuple(0 for _ in shape))

    return pl.pallas_call(
        _final_body,
        grid=grid,
        in_specs=[
            row_spec, row_spec, par_spec, par_spec,
            full((1, D)),
            full((D, 32)), full((1, 32)),
            full((32, 16)), full((1, 16)),
            full((1, 16)), full((1, 1)),
        ],
        out_specs=pl.BlockSpec((BLK,), lambda i: (i,)),
        out_shape=jax.ShapeDtypeStruct((B,), jnp.float32),
    )(u_rows, i_rows, pu, pi, b1, W2, b2, W3, b3, wm, bo)


def kernel(x, gmf_user, gmf_item, mlp_user, mlp_item,
           W1, b1, W2, b2, W3, b3, W_out, b_out):
    diag_wg = jnp.diag(W_out[:D, 0]).astype(jnp.bfloat16)
    eye = jnp.eye(D, dtype=jnp.bfloat16)
    u_tab = _tc_transform(gmf_user.T, mlp_user.T, diag_wg,
                          W1[:D].astype(jnp.bfloat16))
    i_tab = _tc_transform(gmf_item.T, mlp_item.T, eye,
                          W1[D:].astype(jnp.bfloat16))
    uid = x[:, 0]
    iid = x[:, 1]
    uh = (uid >> 1).reshape(NW * NSUB, SUB)
    ih = (iid >> 1).reshape(NW * NSUB, SUB)
    pu = (uid & 1).reshape(B, 1)
    pi = (iid & 1).reshape(B, 1)
    u_rows, i_rows = _sc_gather_kernel()(uh, ih, u_tab, i_tab)
    return _tc_final(
        u_rows, i_rows, pu, pi,
        b1.reshape(1, D), W2, b2.reshape(1, 32), W3, b3.reshape(1, 16),
        W_out[D:, 0].reshape(1, 16), b_out.reshape(1, 1))
